# baseline (speedup 1.0000x reference)
import functools

import jax
import jax.numpy as jnp
from jax import lax
from jax.experimental import pallas as pl
from jax.experimental.pallas import tpu as pltpu
from jax.experimental.pallas import tpu_sc as plsc

B, H, W, C = 4, 384, 384, 96
HW = H * W
NC, NS, L = 2, 16, 16
NW = NC * NS
PER_W = HW // NW
CH = 128
N_CHUNKS = PER_W // CH
N_PAIRS = N_CHUNKS // 2


def _prep_body(fy_ref, fx_ref, aux_ref):
    y = lax.broadcasted_iota(jnp.int32, (H, W), 0)
    x = lax.broadcasted_iota(jnp.int32, (H, W), 1)
    qy = y.astype(jnp.float32) - fy_ref[0]
    qx = x.astype(jnp.float32) - fx_ref[0]
    fy = jnp.clip(jnp.floor(qy), 0.0, H - 2)
    fx = jnp.clip(jnp.floor(qx), 0.0, W - 2)
    ay = jnp.clip(qy - fy, 0.0, 1.0)
    ax = jnp.clip(qx - fx, 0.0, 1.0)
    tl = fy.astype(jnp.int32) * W + fx.astype(jnp.int32)
    aux_ref[0, 0] = tl
    aux_ref[0, 1] = tl + 1
    aux_ref[0, 2] = tl + W
    aux_ref[0, 3] = tl + W + 1
    aux_ref[0, 4] = lax.bitcast_convert_type(ax, jnp.int32)
    aux_ref[0, 5] = lax.bitcast_convert_type(ay, jnp.int32)


_prep = pl.pallas_call(
    _prep_body,
    grid=(B,),
    in_specs=[pl.BlockSpec((1, H, W), lambda i: (i, 0, 0))] * 2,
    out_specs=pl.BlockSpec((1, 6, H, W), lambda i: (i, 0, 0, 0)),
    out_shape=jax.ShapeDtypeStruct((B, 6, H, W), jnp.int32),
)

_mesh = plsc.VectorSubcoreMesh(core_axis_name="c", subcore_axis_name="s")


@functools.partial(
    pl.kernel,
    out_type=jax.ShapeDtypeStruct((HW, C), jnp.float32),
    mesh=_mesh,
    scratch_types=[pltpu.VMEM((6, CH), jnp.int32)] * 2
    + [pltpu.VMEM((CH, C), jnp.float32)] * 10
    + [pltpu.SemaphoreType.DMA] * 6,
    compiler_params=pltpu.CompilerParams(
        needs_layout_passes=False, use_tc_tiling_on_sc=False),
)
def _warp_sc(img, aux, out,
             abuf0, abuf1,
             g00, g01, g02, g03, g10, g11, g12, g13,
             obuf0, obuf1,
             asem0, asem1, gsem0, gsem1, osem0, osem1):
    wid = lax.axis_index("s") * NC + lax.axis_index("c")
    base_w = wid * PER_W
    gset = ((g00, g01, g02, g03), (g10, g11, g12, g13))
    abufs = (abuf0, abuf1)
    gsems = (gsem0, gsem1)

    def aux_slice(ci):
        return aux.at[:, pl.ds(base_w + ci * CH, CH)]

    def out_slice(ci):
        return out.at[pl.ds(base_w + ci * CH, CH)]

    def fire_aux(ci, par):
        return pltpu.async_copy(aux_slice(ci), abufs[par], (asem0, asem1)[par])

    def fire_gathers(par):
        for j in range(4):
            pltpu.async_copy(img.at[abufs[par].at[j]], gset[par][j],
                             gsems[par])

    def wait_gathers(par):
        for j in range(4):
            pltpu.make_async_copy(img.at[abufs[par].at[j]], gset[par][j],
                                  gsems[par]).wait()

    def wait_aux(par):
        pltpu.make_async_copy(aux_slice(0), abufs[par],
                              (asem0, asem1)[par]).wait()

    def wait_out(par, ob):
        pltpu.make_async_copy(out_slice(0), ob, (osem0, osem1)[par]).wait()

    def compute(par, ob):
        ab = abufs[par]
        bufs = gset[par]

        def px(p):
            pv = jnp.zeros((L,), jnp.int32) + p
            axs = plsc.bitcast(
                plsc.load_gather(ab, [jnp.full((L,), 4, jnp.int32), pv]),
                jnp.float32)
            ays = plsc.bitcast(
                plsc.load_gather(ab, [jnp.full((L,), 5, jnp.int32), pv]),
                jnp.float32)
            for j in range(C // L):
                s = pl.ds(j * L, L)
                tl = bufs[0][p, s]
                tr = bufs[1][p, s]
                bl = bufs[2][p, s]
                br = bufs[3][p, s]
                top = tl + axs * (tr - tl)
                bot = bl + axs * (br - bl)
                ob[p, s] = top + ays * (bot - top)

        plsc.parallel_loop(0, CH, 1, unroll=4)(px)

    fire_aux(0, 0).wait()
    fire_gathers(0)
    fire_aux(1, 1)

    def pair(pi, carry):
        wait_aux(1)
        fire_gathers(1)
        wait_gathers(0)

        @pl.when(pi >= 1)
        def _():
            wait_out(0, obuf0)

        compute(0, obuf0)
        pltpu.async_copy(obuf0, out_slice(2 * pi), osem0)

        @pl.when(pi < N_PAIRS - 1)
        def _():
            fire_aux(2 * pi + 2, 0)

        @pl.when(pi < N_PAIRS - 1)
        def _():
            wait_aux(0)
            fire_gathers(0)

        wait_gathers(1)

        @pl.when(pi >= 1)
        def _():
            wait_out(1, obuf1)

        compute(1, obuf1)
        pltpu.async_copy(obuf1, out_slice(2 * pi + 1), osem1)

        @pl.when(pi < N_PAIRS - 1)
        def _():
            fire_aux(2 * pi + 3, 1)

        return carry

    lax.fori_loop(0, N_PAIRS, pair, 0)
    wait_out(0, obuf0)
    wait_out(1, obuf1)


def kernel(image, flow):
    aux = _prep(flow[..., 0], flow[..., 1])
    outs = [
        _warp_sc(image[b].reshape(HW, C), aux[b].reshape(6, HW))
        for b in range(B)
    ]
    return jnp.stack(outs).reshape(B, H, W, C)


# speedup vs baseline: 1.3229x; 1.3229x over previous
import functools

import jax
import jax.numpy as jnp
from jax import lax
from jax.experimental import pallas as pl
from jax.experimental.pallas import tpu as pltpu
from jax.experimental.pallas import tpu_sc as plsc

B, H, W, C = 4, 384, 384, 96
CP = 128
P = B * H * W
NC, NS, L = 2, 16, 16
NW = NC * NS
PER_W = P // NW
CH = 96
N_CHUNKS = PER_W // CH
N_PAIRS = N_CHUNKS // 2

R_BLK = 256


def _prep_body(fy_ref, fx_ref, aux_ref):
    pid = pl.program_id(0)
    rows = lax.broadcasted_iota(jnp.int32, (R_BLK, W), 0) + pid * R_BLK
    cols = lax.broadcasted_iota(jnp.int32, (R_BLK, W), 1)
    y = rows % H
    b = rows // H
    qy = y.astype(jnp.float32) - fy_ref[...]
    qx = cols.astype(jnp.float32) - fx_ref[...]
    fy = jnp.clip(jnp.floor(qy), 0.0, H - 2)
    fx = jnp.clip(jnp.floor(qx), 0.0, W - 2)
    ay = jnp.clip(qy - fy, 0.0, 1.0)
    ax = jnp.clip(qx - fx, 0.0, 1.0)
    tl = b * (H * W) + fy.astype(jnp.int32) * W + fx.astype(jnp.int32)
    aux_ref[0] = tl
    aux_ref[1] = tl + 1
    aux_ref[2] = tl + W
    aux_ref[3] = tl + W + 1
    aux_ref[4] = lax.bitcast_convert_type(ax, jnp.int32)
    aux_ref[5] = lax.bitcast_convert_type(ay, jnp.int32)


_prep = pl.pallas_call(
    _prep_body,
    grid=(B * H // R_BLK,),
    in_specs=[pl.BlockSpec((R_BLK, W), lambda i: (i, 0))] * 2,
    out_specs=pl.BlockSpec((6, R_BLK, W), lambda i: (0, i, 0)),
    out_shape=jax.ShapeDtypeStruct((6, B * H, W), jnp.int32),
)

_mesh = plsc.VectorSubcoreMesh(core_axis_name="c", subcore_axis_name="s")


@functools.partial(
    pl.kernel,
    out_type=jax.ShapeDtypeStruct((P, CP), jnp.float32),
    mesh=_mesh,
    scratch_types=[pltpu.VMEM((6, CH), jnp.int32)] * 2
    + [pltpu.VMEM((CH, CP), jnp.float32)] * 8
    + [pltpu.VMEM((CH, C), jnp.float32)] * 2
    + [pltpu.SemaphoreType.DMA] * 6,
    compiler_params=pltpu.CompilerParams(
        needs_layout_passes=False, use_tc_tiling_on_sc=False),
)
def _warp_sc(img, aux, out,
             abuf0, abuf1,
             g00, g01, g02, g03, g10, g11, g12, g13,
             obuf0, obuf1,
             asem0, asem1, gsem0, gsem1, osem0, osem1):
    wid = lax.axis_index("s") * NC + lax.axis_index("c")
    base_w = wid * PER_W
    gset = ((g00, g01, g02, g03), (g10, g11, g12, g13))
    abufs = (abuf0, abuf1)
    gsems = (gsem0, gsem1)

    def aux_slice(ci):
        return aux.at[:, pl.ds(base_w + ci * CH, CH)]

    def out_slice(ci):
        return out.at[pl.ds(base_w + ci * CH, CH), pl.ds(0, C)]

    def fire_aux(ci, par):
        return pltpu.async_copy(aux_slice(ci), abufs[par], (asem0, asem1)[par])

    def fire_gathers(par):
        for j in range(4):
            pltpu.async_copy(img.at[abufs[par].at[j]], gset[par][j],
                             gsems[par])

    def wait_gathers(par):
        for j in range(4):
            pltpu.make_async_copy(img.at[abufs[par].at[j]], gset[par][j],
                                  gsems[par]).wait()

    def wait_aux(par):
        pltpu.make_async_copy(aux_slice(0), abufs[par],
                              (asem0, asem1)[par]).wait()

    def wait_out(par, ob):
        pltpu.make_async_copy(out_slice(0), ob, (osem0, osem1)[par]).wait()

    def compute(par, ob):
        ab = abufs[par]
        bufs = gset[par]

        def px(p):
            pv = jnp.zeros((L,), jnp.int32) + p
            axs = plsc.bitcast(
                plsc.load_gather(ab, [jnp.full((L,), 4, jnp.int32), pv]),
                jnp.float32)
            ays = plsc.bitcast(
                plsc.load_gather(ab, [jnp.full((L,), 5, jnp.int32), pv]),
                jnp.float32)
            for j in range(C // L):
                s = pl.ds(j * L, L)
                tl = bufs[0][p, s]
                tr = bufs[1][p, s]
                bl = bufs[2][p, s]
                br = bufs[3][p, s]
                top = tl + axs * (tr - tl)
                bot = bl + axs * (br - bl)
                ob[p, s] = top + ays * (bot - top)

        plsc.parallel_loop(0, CH, 1, unroll=4)(px)

    fire_aux(0, 0).wait()
    fire_gathers(0)
    fire_aux(1, 1)

    def pair(pi, carry):
        wait_aux(1)
        fire_gathers(1)
        wait_gathers(0)

        @pl.when(pi >= 1)
        def _():
            wait_out(0, obuf0)

        compute(0, obuf0)
        pltpu.async_copy(obuf0, out_slice(2 * pi), osem0)

        @pl.when(pi < N_PAIRS - 1)
        def _():
            fire_aux(2 * pi + 2, 0)

        @pl.when(pi < N_PAIRS - 1)
        def _():
            wait_aux(0)
            fire_gathers(0)

        wait_gathers(1)

        @pl.when(pi >= 1)
        def _():
            wait_out(1, obuf1)

        compute(1, obuf1)
        pltpu.async_copy(obuf1, out_slice(2 * pi + 1), osem1)

        @pl.when(pi < N_PAIRS - 1)
        def _():
            fire_aux(2 * pi + 3, 1)

        return carry

    lax.fori_loop(0, N_PAIRS, pair, 0)
    wait_out(0, obuf0)
    wait_out(1, obuf1)


def kernel(image, flow):
    flow_y = flow[..., 0].reshape(B * H, W)
    flow_x = flow[..., 1].reshape(B * H, W)
    aux = _prep(flow_y, flow_x)
    table = jnp.pad(image, ((0, 0), (0, 0), (0, 0), (0, CP - C)))
    out = _warp_sc(table.reshape(P, CP), aux.reshape(6, P))
    return out[:, :C].reshape(B, H, W, C)
